# baseline (speedup 1.0000x reference)
import functools

import jax
import jax.numpy as jnp
from jax import lax
from jax.experimental import pallas as pl
from jax.experimental.pallas import tpu as pltpu
from jax.experimental.pallas import tpu_sc as plsc

_NC = 2
_NS = 16
_L = 16
_NW = _NC * _NS

_STREAM = 128


def _matvec(table, w_row):
    vocab, emb = table.shape
    blk = 16384
    nb = 4
    steps = (vocab // blk) // nb
    obk = blk // 128

    def body(hbm_ref, w_ref, o_ref, b0, b1, b2, b3, s0, s1, s2, s3):
        bufs = (b0, b1, b2, b3)
        sems = (s0, s1, s2, s3)
        i = pl.program_id(0)

        def copy_in(block_idx, k):
            return pltpu.make_async_copy(
                hbm_ref.at[pl.ds(block_idx * blk, blk), :], bufs[k], sems[k])

        @pl.when(i == 0)
        def _():
            for k in range(nb):
                copy_in(k, k).start()

        for k in range(nb):
            copy_in(i * nb + k, k).wait()
            s = jnp.sum(bufs[k][...] * w_ref[0:1, :], axis=1)
            o_ref[pl.ds(k * obk, obk), :] = s.reshape(obk, 128)

            @pl.when(i + 1 < steps)
            def _():
                copy_in((i + 1) * nb + k, k).start()

    outs = pl.pallas_call(
        body,
        grid=(steps,),
        in_specs=[
            pl.BlockSpec(memory_space=pl.ANY),
            pl.BlockSpec((8, emb), lambda i: (0, 0)),
        ],
        out_specs=pl.BlockSpec((nb * obk, 128), lambda i: (i, 0)),
        out_shape=jax.ShapeDtypeStruct((steps * nb * obk, 128), jnp.float32),
        scratch_shapes=[pltpu.VMEM((blk, emb), jnp.float32)] * nb
        + [pltpu.SemaphoreType.DMA] * nb,
    )(table, w_row)
    outs = [outs]

    done = nb * steps * blk
    tail_rows = vocab - done
    tail_grid = pl.cdiv(tail_rows, blk)

    def tail_body(tbl_ref, w_ref, o_ref):
        s = jnp.sum(tbl_ref[...] * w_ref[0:1, :], axis=1)
        o_ref[...] = s.reshape(blk // 128, 128)

    tail = pl.pallas_call(
        tail_body,
        grid=(tail_grid,),
        in_specs=[
            pl.BlockSpec((blk, emb), lambda i: (i, 0)),
            pl.BlockSpec((8, emb), lambda i: (0, 0)),
        ],
        out_specs=pl.BlockSpec((blk // 128, 128), lambda i: (i, 0)),
        out_shape=jax.ShapeDtypeStruct((tail_grid * blk // 128, 128), jnp.float32),
    )(lax.slice_in_dim(table, done, vocab), w_row)

    return jnp.concatenate(outs + [tail], axis=0)


def _make_sc_gather(batch, hist, vocab):
    rpw = batch // _NW
    groups = rpw // _L
    chunk = hist * _L
    nstr = chunk // _STREAM

    mesh = plsc.VectorSubcoreMesh(core_axis_name="c", subcore_axis_name="s")

    unroll = 8

    @functools.partial(
        pl.kernel,
        out_type=jax.ShapeDtypeStruct((batch,), jnp.float32),
        mesh=mesh,
        scratch_types=[
            pltpu.VMEM((groups * nstr, _STREAM), jnp.int32),
            pltpu.VMEM((chunk,), jnp.float32),
            pltpu.VMEM((chunk,), jnp.float32),
            pltpu.VMEM((rpw,), jnp.float32),
            pltpu.VMEM((_L,), jnp.float32),
            pltpu.SemaphoreType.DMA,
            pltpu.SemaphoreType.DMA,
        ],
    )
    def sc_kernel(t_hbm, xt_hbm, b_hbm, out_hbm,
                  idx_v, vals_a, vals_b, res_v, b_v, sem_a, sem_b):
        wid = lax.axis_index("s") * _NC + lax.axis_index("c")
        pltpu.sync_copy(b_hbm, b_v)
        pltpu.sync_copy(xt_hbm.at[wid], idx_v)
        bvec = b_v[...]

        def fire(g, buf, sem):
            for j in range(nstr):
                pltpu.async_copy(
                    t_hbm.at[idx_v.at[g * nstr + j]],
                    buf.at[pl.ds(j * _STREAM, _STREAM)],
                    sem)

        def drain(buf, sem):
            d = pltpu.make_async_copy(
                t_hbm.at[idx_v.at[0]], buf.at[pl.ds(0, _STREAM)], sem)
            for _ in range(nstr):
                d.wait()

        def accumulate(buf, g):
            def acc_body(k, acc):
                base = k * (unroll * _L)
                for u in range(unroll):
                    acc = acc + buf[pl.ds(base + u * _L, _L)]
                return acc
            s = lax.fori_loop(0, chunk // (unroll * _L), acc_body,
                              jnp.zeros((_L,), jnp.float32))
            z = s * (1.0 / hist) + bvec
            e = jnp.exp(z + z)
            res_v[pl.ds(g * _L, _L)] = 1.0 - 2.0 / (e + 1.0)

        fire(0, vals_a, sem_a)

        def pair(i, carry):
            g_even = 2 * i
            g_odd = g_even + 1
            fire(g_odd, vals_b, sem_b)
            drain(vals_a, sem_a)
            accumulate(vals_a, g_even)

            @pl.when(g_odd + 1 < groups)
            def _():
                fire(g_odd + 1, vals_a, sem_a)
            drain(vals_b, sem_b)
            accumulate(vals_b, g_odd)
            return carry

        lax.fori_loop(0, groups // 2, pair, 0)
        pltpu.sync_copy(res_v, out_hbm.at[pl.ds(wid * rpw, rpw)])

    return sc_kernel


def kernel(x, table, W, b):
    batch, hist = x.shape
    vocab, emb = table.shape

    w_row = jnp.broadcast_to(W.reshape(1, emb), (8, emb)).astype(jnp.float32)
    t2d = _matvec(table, w_row)
    t = t2d.reshape(t2d.shape[0] * 128)

    rpw = batch // _NW
    groups = rpw // _L
    xt = x.astype(jnp.int32).reshape(_NW, groups, _L, hist)
    xt = xt.transpose(0, 1, 3, 2).reshape(_NW, groups * hist * _L // _STREAM, _STREAM)

    b16 = jnp.broadcast_to(b.astype(jnp.float32), (_L,))

    out = _make_sc_gather(batch, hist, vocab)(t, xt, b16)
    return out.reshape(batch, 1)


# speedup vs baseline: 1.0010x; 1.0010x over previous
import functools

import jax
import jax.numpy as jnp
from jax import lax
from jax.experimental import pallas as pl
from jax.experimental.pallas import tpu as pltpu
from jax.experimental.pallas import tpu_sc as plsc

_NC = 2
_NS = 16
_L = 16
_NW = _NC * _NS

_STREAM = 128


def _matvec(table, w_row):
    vocab, emb = table.shape
    blk = 8192
    nb = 4
    steps = (vocab // blk) // nb
    obk = blk // 128

    def body(hbm_ref, w_ref, o_ref, b0, b1, b2, b3, s0, s1, s2, s3):
        bufs = (b0, b1, b2, b3)
        sems = (s0, s1, s2, s3)
        i = pl.program_id(0)

        def copy_in(block_idx, k):
            return pltpu.make_async_copy(
                hbm_ref.at[pl.ds(block_idx * blk, blk), :], bufs[k], sems[k])

        @pl.when(i == 0)
        def _():
            for k in range(nb):
                copy_in(k, k).start()

        for k in range(nb):
            copy_in(i * nb + k, k).wait()
            s = jnp.sum(bufs[k][...] * w_ref[0:1, :], axis=1)
            o_ref[pl.ds(k * obk, obk), :] = s.reshape(obk, 128)

            @pl.when(i + 1 < steps)
            def _():
                copy_in((i + 1) * nb + k, k).start()

    outs = pl.pallas_call(
        body,
        grid=(steps,),
        in_specs=[
            pl.BlockSpec(memory_space=pl.ANY),
            pl.BlockSpec((8, emb), lambda i: (0, 0)),
        ],
        out_specs=pl.BlockSpec((nb * obk, 128), lambda i: (i, 0)),
        out_shape=jax.ShapeDtypeStruct((steps * nb * obk, 128), jnp.float32),
        scratch_shapes=[pltpu.VMEM((blk, emb), jnp.float32)] * nb
        + [pltpu.SemaphoreType.DMA] * nb,
    )(table, w_row)
    outs = [outs]

    done = nb * steps * blk
    tail_rows = vocab - done
    tail_grid = pl.cdiv(tail_rows, blk)

    def tail_body(tbl_ref, w_ref, o_ref):
        s = jnp.sum(tbl_ref[...] * w_ref[0:1, :], axis=1)
        o_ref[...] = s.reshape(blk // 128, 128)

    tail = pl.pallas_call(
        tail_body,
        grid=(tail_grid,),
        in_specs=[
            pl.BlockSpec((blk, emb), lambda i: (i, 0)),
            pl.BlockSpec((8, emb), lambda i: (0, 0)),
        ],
        out_specs=pl.BlockSpec((blk // 128, 128), lambda i: (i, 0)),
        out_shape=jax.ShapeDtypeStruct((tail_grid * blk // 128, 128), jnp.float32),
    )(lax.slice_in_dim(table, done, vocab), w_row)

    return jnp.concatenate(outs + [tail], axis=0)


def _make_sc_gather(batch, hist, vocab):
    rpw = batch // _NW
    groups = rpw // _L
    chunk = hist * _L
    nstr = chunk // _STREAM

    mesh = plsc.VectorSubcoreMesh(core_axis_name="c", subcore_axis_name="s")

    unroll = 8

    @functools.partial(
        pl.kernel,
        out_type=jax.ShapeDtypeStruct((batch,), jnp.float32),
        mesh=mesh,
        scratch_types=[
            pltpu.VMEM((groups * nstr, _STREAM), jnp.int32),
            pltpu.VMEM((chunk,), jnp.float32),
            pltpu.VMEM((chunk,), jnp.float32),
            pltpu.VMEM((rpw,), jnp.float32),
            pltpu.VMEM((_L,), jnp.float32),
            pltpu.SemaphoreType.DMA,
            pltpu.SemaphoreType.DMA,
        ],
    )
    def sc_kernel(t_hbm, xt_hbm, b_hbm, out_hbm,
                  idx_v, vals_a, vals_b, res_v, b_v, sem_a, sem_b):
        wid = lax.axis_index("s") * _NC + lax.axis_index("c")
        pltpu.sync_copy(b_hbm, b_v)
        pltpu.sync_copy(xt_hbm.at[wid], idx_v)
        bvec = b_v[...]

        def fire(g, buf, sem):
            for j in range(nstr):
                pltpu.async_copy(
                    t_hbm.at[idx_v.at[g * nstr + j]],
                    buf.at[pl.ds(j * _STREAM, _STREAM)],
                    sem)

        def drain(buf, sem):
            d = pltpu.make_async_copy(
                t_hbm.at[idx_v.at[0]], buf.at[pl.ds(0, _STREAM)], sem)
            for _ in range(nstr):
                d.wait()

        def accumulate(buf, g):
            def acc_body(k, acc):
                base = k * (unroll * _L)
                for u in range(unroll):
                    acc = acc + buf[pl.ds(base + u * _L, _L)]
                return acc
            s = lax.fori_loop(0, chunk // (unroll * _L), acc_body,
                              jnp.zeros((_L,), jnp.float32))
            z = s * (1.0 / hist) + bvec
            e = jnp.exp(z + z)
            res_v[pl.ds(g * _L, _L)] = 1.0 - 2.0 / (e + 1.0)

        fire(0, vals_a, sem_a)

        def pair(i, carry):
            g_even = 2 * i
            g_odd = g_even + 1
            fire(g_odd, vals_b, sem_b)
            drain(vals_a, sem_a)
            accumulate(vals_a, g_even)

            @pl.when(g_odd + 1 < groups)
            def _():
                fire(g_odd + 1, vals_a, sem_a)
            drain(vals_b, sem_b)
            accumulate(vals_b, g_odd)
            return carry

        lax.fori_loop(0, groups // 2, pair, 0)
        pltpu.sync_copy(res_v, out_hbm.at[pl.ds(wid * rpw, rpw)])

    return sc_kernel


def kernel(x, table, W, b):
    batch, hist = x.shape
    vocab, emb = table.shape

    w_row = jnp.broadcast_to(W.reshape(1, emb), (8, emb)).astype(jnp.float32)
    t2d = _matvec(table, w_row)
    t = t2d.reshape(t2d.shape[0] * 128)

    rpw = batch // _NW
    groups = rpw // _L
    xt = x.astype(jnp.int32).reshape(_NW, groups, _L, hist)
    xt = xt.transpose(0, 1, 3, 2).reshape(_NW, groups * hist * _L // _STREAM, _STREAM)

    b16 = jnp.broadcast_to(b.astype(jnp.float32), (_L,))

    out = _make_sc_gather(batch, hist, vocab)(t, xt, b16)
    return out.reshape(batch, 1)
